# baseline (speedup 1.0000x reference)
import functools

import jax
import jax.numpy as jnp
import numpy as np
from jax import lax
from jax.experimental import pallas as pl
from jax.experimental.pallas import tpu as pltpu
from jax.experimental.pallas import tpu_sc as plsc

B = 16
C = 16
N = 131072
HALF = N // 2
NVEC = HALF // 16
KS = 500
TIECAP = 544
CH = 1024
NGRP = HALF // CH

MASK31 = np.int32(0x7FFFFFFF)
IMIN = np.int32(-2147483648)
SCALE = np.float32(0.625)


def _keys(v):
    iv = plsc.bitcast(v, jnp.int32)
    ks = iv ^ (lax.shift_right_arithmetic(iv, np.int32(31)) & MASK31)
    ub = ks ^ IMIN
    return ks, ub


def _sc_body(x_hbm, out_hbm, bh_hbm, bc_hbm, acc, buf_a, buf_b, hist, hist2,
             tie_idx, stage, sem_a, sem_b):
    cid = lax.axis_index("c")
    sid = lax.axis_index("s")
    b = cid * 8 + sid // 2
    h = sid % 2
    wid = cid * 16 + sid
    pid = cid * 16 + (sid ^ 1)
    base_off = h * HALF

    iota16 = lax.iota(jnp.int32, 16)
    ones16 = jnp.ones((16,), jnp.int32)

    def _grp_copies(buf, sem, g):
        off = base_off + g * CH
        return [pltpu.make_async_copy(x_hbm.at[b, c, pl.ds(off, CH)],
                                      buf.at[c], sem)
                for c in range(C)]

    def _fire(buf, sem, g):
        for cp in _grp_copies(buf, sem, g):
            cp.start()

    def _drain(buf, sem, g):
        for cp in _grp_copies(buf, sem, g):
            cp.wait()

    def _accum(buf, g):
        base = g * CH

        def body(i, _):
            lo = i * 16
            s = buf[0, pl.ds(lo, 16)]
            for c in range(1, C):
                s = s + buf[c, pl.ds(lo, 16)]
            acc[pl.ds(base + lo, 16)] = s
            return 0

        lax.fori_loop(0, CH // 16, body, 0, unroll=2)

    _fire(buf_a, sem_a, 0)

    def mean_step(u, _):
        g0 = u * 2
        g1 = u * 2 + 1
        _drain(buf_a, sem_a, g0)
        _fire(buf_b, sem_b, g1)
        _accum(buf_a, g0)
        _drain(buf_b, sem_b, g1)

        @pl.when(u < NGRP // 2 - 1)
        def _():
            _fire(buf_a, sem_a, g1 + 1)

        _accum(buf_b, g1)
        return 0

    lax.fori_loop(0, NGRP // 2, mean_step, 0)

    zeros16 = jnp.zeros((16,), jnp.int32)
    lane_base = iota16 * 256

    def _zero_hist():
        def zb(i, _):
            hist[pl.ds(i * 16, 16)] = zeros16
            return 0
        lax.fori_loop(0, 256, zb, 0)

    def _fold_merge_pick(target):
        def fold(i, _):
            s = hist[pl.ds(i * 16, 16)]
            for l in range(1, 16):
                s = s + hist[pl.ds(l * 256 + i * 16, 16)]
            hist2[pl.ds(i * 16, 16)] = s
            return 0

        lax.fori_loop(0, 16, fold, 0)
        pltpu.sync_copy(hist2.at[pl.ds(0, 256)], bh_hbm.at[wid])
        plsc.subcore_barrier()
        pltpu.sync_copy(bh_hbm.at[pid], hist2.at[pl.ds(256, 256)])
        plsc.subcore_barrier()

        def scan(i, carry):
            above, bsel, gsel = carry
            blk = 15 - i
            hv = (hist2[pl.ds(blk * 16, 16)]
                  + hist2[pl.ds(256 + blk * 16, 16)])
            rev = lax.rev(hv, (0,))
            csum = plsc.cumsum(rev)
            ca = above + csum - rev
            m = (ca < target) & (ca + rev >= target)
            binv = blk * 16 + 15 - iota16
            bsel = bsel + jnp.sum(jnp.where(m, binv, 0))
            gsel = gsel + jnp.sum(jnp.where(m, ca, 0))
            above = above + jnp.sum(hv)
            return (above, bsel, gsel)

        _, bsel, gsel = lax.fori_loop(0, 16, scan,
                                      (jnp.int32(0), jnp.int32(0), jnp.int32(0)))
        return bsel, gsel

    prefix = jnp.int32(0)
    g_tot = jnp.int32(0)
    for level in range(4):
        shift = np.int32(24 - 8 * level)
        _zero_hist()

        if level == 0:
            def pL(t, _, shift=shift):
                v = acc[pl.ds(t * 16, 16)]
                _, ub = _keys(v)
                binv = lax.shift_right_logical(ub, shift) & np.int32(0xFF)
                plsc.addupdate_scatter(hist, [lane_base + binv], ones16)
                return 0
        else:
            def pL(t, _, shift=shift, prefix=prefix):
                v = acc[pl.ds(t * 16, 16)]
                _, ub = _keys(v)
                m = lax.shift_right_logical(ub, shift + 8) == prefix
                binv = lax.shift_right_logical(ub, shift) & np.int32(0xFF)
                plsc.addupdate_scatter(hist, [lane_base + binv], ones16, mask=m)
                return 0

        lax.fori_loop(0, NVEC, pL, 0, unroll=2)
        bsel, gsel = _fold_merge_pick(jnp.int32(KS) - g_tot)
        prefix = lax.shift_left(prefix, np.int32(8)) | bsel
        g_tot = g_tot + gsel

    ub_thr = prefix
    ks_thr = ub_thr ^ IMIN
    need = jnp.int32(KS) - g_tot
    i_thr = ks_thr ^ (lax.shift_right_arithmetic(ks_thr, np.int32(31)) & MASK31)
    v_thr_out = plsc.bitcast(jnp.full((16,), i_thr, jnp.int32), jnp.float32) * SCALE

    def fmask(t, cnt):
        v = acc[pl.ds(t * 16, 16)]
        ks, _ = _keys(v)
        m_gt = ks > ks_thr
        acc[pl.ds(t * 16, 16)] = jnp.where(m_gt, v * SCALE, np.float32(0.0))
        m_eq = ks == ks_thr
        rank = plsc.cumsum(m_eq.astype(jnp.int32))
        tgt = cnt + rank - 1
        mw = m_eq & (tgt < np.int32(TIECAP))
        plsc.store_scatter(tie_idx, [tgt], t * 16 + iota16, mask=mw)
        return cnt + jnp.sum(m_eq.astype(jnp.int32))

    tcnt = lax.fori_loop(0, NVEC, fmask, jnp.int32(0), unroll=2)

    stage[...] = jnp.full((16,), tcnt, jnp.int32)
    pltpu.sync_copy(stage, bc_hbm.at[wid])
    plsc.subcore_barrier()
    pltpu.sync_copy(bc_hbm.at[pid], stage)
    t_other = jnp.max(stage[...], axis=0)
    t_first = jnp.where(h == 0, tcnt, t_other)
    quota = jnp.clip(need - h * t_first, 0,
                     jnp.minimum(tcnt, jnp.int32(TIECAP)))

    def sties(t, _):
        ivec = tie_idx[pl.ds(t * 16, 16)]
        pos = t * 16 + iota16
        m = pos < quota
        plsc.store_scatter(acc, [ivec], v_thr_out, mask=m)
        return 0

    lax.fori_loop(0, TIECAP // 16, sties, 0)

    pltpu.sync_copy(acc, out_hbm.at[b, pl.ds(base_off, HALF)])


def kernel(x):
    x3 = x.reshape(B, C, N)
    mesh = plsc.VectorSubcoreMesh(core_axis_name="c", subcore_axis_name="s")
    run = pl.kernel(
        _sc_body,
        out_type=(
            jax.ShapeDtypeStruct((B, N), jnp.float32),
            jax.ShapeDtypeStruct((32, 256), jnp.int32),
            jax.ShapeDtypeStruct((32, 16), jnp.int32),
        ),
        mesh=mesh,
        scratch_types=[
            pltpu.VMEM((HALF,), jnp.float32),
            pltpu.VMEM((C, CH), jnp.float32),
            pltpu.VMEM((C, CH), jnp.float32),
            pltpu.VMEM((4096,), jnp.int32),
            pltpu.VMEM((512,), jnp.int32),
            pltpu.VMEM((TIECAP,), jnp.int32),
            pltpu.VMEM((16,), jnp.int32),
            pltpu.SemaphoreType.DMA,
            pltpu.SemaphoreType.DMA,
        ],
        compiler_params=pltpu.CompilerParams(needs_layout_passes=False),
    )
    mask, _, _ = run(x3)
    return mask.reshape(B, N, 1)


# speedup vs baseline: 1.1875x; 1.1875x over previous
import functools

import jax
import jax.numpy as jnp
import numpy as np
from jax import lax
from jax.experimental import pallas as pl
from jax.experimental.pallas import tpu as pltpu
from jax.experimental.pallas import tpu_sc as plsc

B = 16
C = 16
N = 131072
HALF = N // 2
NVEC = HALF // 16
KS = 500
TIECAP = 544
CH = 1024
NGRP = HALF // CH

MASK31 = np.int32(0x7FFFFFFF)
IMIN = np.int32(-2147483648)
SCALE = np.float32(0.625)


def _keys(v):
    iv = plsc.bitcast(v, jnp.int32)
    ks = iv ^ (lax.shift_right_arithmetic(iv, np.int32(31)) & MASK31)
    ub = ks ^ IMIN
    return ks, ub


def _sc_body(x_hbm, out_hbm, bh_hbm, bc_hbm, acc, buf_a, buf_b, hist, hist2,
             tie_idx, stage, sem_a, sem_b):
    cid = lax.axis_index("c")
    sid = lax.axis_index("s")
    b = cid * 8 + sid // 2
    h = sid % 2
    wid = cid * 16 + sid
    pid = cid * 16 + (sid ^ 1)
    base_off = h * HALF

    iota16 = lax.iota(jnp.int32, 16)
    ones16 = jnp.ones((16,), jnp.int32)

    def _grp_copies(buf, sem, g):
        r0 = pl.multiple_of((base_off + g * CH) // 128, 8)
        return [pltpu.make_async_copy(x_hbm.at[b, c, pl.ds(r0, CH // 128)],
                                      buf.at[c], sem)
                for c in range(C)]

    def _fire(buf, sem, g):
        for cp in _grp_copies(buf, sem, g):
            cp.start()

    def _drain(buf, sem, g):
        for cp in _grp_copies(buf, sem, g):
            cp.wait()

    def _accum(buf, g):
        def body(i, _):
            r = i // 8
            lo = (i % 8) * 16
            s = buf[0, r, pl.ds(lo, 16)]
            for c in range(1, C):
                s = s + buf[c, r, pl.ds(lo, 16)]
            acc[g * 8 + r, pl.ds(lo, 16)] = s
            return 0

        lax.fori_loop(0, CH // 16, body, 0, unroll=2)

    _fire(buf_a, sem_a, 0)

    def mean_step(u, _):
        g0 = u * 2
        g1 = u * 2 + 1
        _drain(buf_a, sem_a, g0)
        _fire(buf_b, sem_b, g1)
        _accum(buf_a, g0)
        _drain(buf_b, sem_b, g1)

        @pl.when(u < NGRP // 2 - 1)
        def _():
            _fire(buf_a, sem_a, g1 + 1)

        _accum(buf_b, g1)
        return 0

    lax.fori_loop(0, NGRP // 2, mean_step, 0)

    zeros16 = jnp.zeros((16,), jnp.int32)
    lane_base = iota16 * 256

    def _zero_hist():
        def zb(i, _):
            hist[pl.ds(i * 16, 16)] = zeros16
            return 0
        lax.fori_loop(0, 256, zb, 0)

    def _fold_merge_pick(target):
        def fold(i, _):
            s = hist[pl.ds(i * 16, 16)]
            for l in range(1, 16):
                s = s + hist[pl.ds(l * 256 + i * 16, 16)]
            hist2[pl.ds(i * 16, 16)] = s
            return 0

        lax.fori_loop(0, 16, fold, 0)
        pltpu.sync_copy(hist2.at[pl.ds(0, 256)], bh_hbm.at[wid])
        plsc.subcore_barrier()
        pltpu.sync_copy(bh_hbm.at[pid], hist2.at[pl.ds(256, 256)])
        plsc.subcore_barrier()

        def scan(i, carry):
            above, bsel, gsel = carry
            blk = 15 - i
            hv = (hist2[pl.ds(blk * 16, 16)]
                  + hist2[pl.ds(256 + blk * 16, 16)])
            rev = lax.rev(hv, (0,))
            csum = plsc.cumsum(rev)
            ca = above + csum - rev
            m = (ca < target) & (ca + rev >= target)
            binv = blk * 16 + 15 - iota16
            bsel = bsel + jnp.sum(jnp.where(m, binv, 0))
            gsel = gsel + jnp.sum(jnp.where(m, ca, 0))
            above = above + jnp.sum(hv)
            return (above, bsel, gsel)

        _, bsel, gsel = lax.fori_loop(0, 16, scan,
                                      (jnp.int32(0), jnp.int32(0), jnp.int32(0)))
        return bsel, gsel

    prefix = jnp.int32(0)
    g_tot = jnp.int32(0)
    for level in range(4):
        shift = np.int32(24 - 8 * level)
        _zero_hist()

        if level == 0:
            def pL(t, _, shift=shift):
                v = acc[t // 8, pl.ds((t % 8) * 16, 16)]
                _, ub = _keys(v)
                binv = lax.shift_right_logical(ub, shift) & np.int32(0xFF)
                plsc.addupdate_scatter(hist, [lane_base + binv], ones16)
                return 0
        else:
            def pL(t, _, shift=shift, prefix=prefix):
                v = acc[t // 8, pl.ds((t % 8) * 16, 16)]
                _, ub = _keys(v)
                m = lax.shift_right_logical(ub, shift + 8) == prefix
                binv = lax.shift_right_logical(ub, shift) & np.int32(0xFF)
                plsc.addupdate_scatter(hist, [lane_base + binv], ones16, mask=m)
                return 0

        lax.fori_loop(0, NVEC, pL, 0, unroll=2)
        bsel, gsel = _fold_merge_pick(jnp.int32(KS) - g_tot)
        prefix = lax.shift_left(prefix, np.int32(8)) | bsel
        g_tot = g_tot + gsel

    ub_thr = prefix
    ks_thr = ub_thr ^ IMIN
    need = jnp.int32(KS) - g_tot
    i_thr = ks_thr ^ (lax.shift_right_arithmetic(ks_thr, np.int32(31)) & MASK31)
    v_thr_out = plsc.bitcast(jnp.full((16,), i_thr, jnp.int32), jnp.float32) * SCALE

    def fmask(t, cnt):
        v = acc[t // 8, pl.ds((t % 8) * 16, 16)]
        ks, _ = _keys(v)
        m_gt = ks > ks_thr
        acc[t // 8, pl.ds((t % 8) * 16, 16)] = jnp.where(m_gt, v * SCALE,
                                                         np.float32(0.0))
        m_eq = ks == ks_thr
        rank = plsc.cumsum(m_eq.astype(jnp.int32))
        tgt = cnt + rank - 1
        mw = m_eq & (tgt < np.int32(TIECAP))
        plsc.store_scatter(tie_idx, [tgt], t * 16 + iota16, mask=mw)
        return cnt + jnp.sum(m_eq.astype(jnp.int32))

    tcnt = lax.fori_loop(0, NVEC, fmask, jnp.int32(0), unroll=2)

    stage[...] = jnp.full((16,), tcnt, jnp.int32)
    pltpu.sync_copy(stage, bc_hbm.at[wid])
    plsc.subcore_barrier()
    pltpu.sync_copy(bc_hbm.at[pid], stage)
    t_other = jnp.max(stage[...], axis=0)
    t_first = jnp.where(h == 0, tcnt, t_other)
    quota = jnp.clip(need - h * t_first, 0,
                     jnp.minimum(tcnt, jnp.int32(TIECAP)))

    def sties(t, _):
        ivec = tie_idx[pl.ds(t * 16, 16)]
        pos = t * 16 + iota16
        m = pos < quota
        rows = lax.shift_right_logical(ivec, np.int32(7))
        lanes = ivec & np.int32(127)
        plsc.store_scatter(acc, [rows, lanes], v_thr_out, mask=m)
        return 0

    lax.fori_loop(0, TIECAP // 16, sties, 0)

    ro = pl.multiple_of(base_off // 128, 8)
    pltpu.sync_copy(acc, out_hbm.at[b, pl.ds(ro, HALF // 128)])


def kernel(x):
    x4 = x.reshape(B, C, N // 128, 128)
    mesh = plsc.VectorSubcoreMesh(core_axis_name="c", subcore_axis_name="s")
    run = pl.kernel(
        _sc_body,
        out_type=(
            jax.ShapeDtypeStruct((B, N // 128, 128), jnp.float32),
            jax.ShapeDtypeStruct((32, 256), jnp.int32),
            jax.ShapeDtypeStruct((32, 16), jnp.int32),
        ),
        mesh=mesh,
        scratch_types=[
            pltpu.VMEM((HALF // 128, 128), jnp.float32),
            pltpu.VMEM((C, CH // 128, 128), jnp.float32),
            pltpu.VMEM((C, CH // 128, 128), jnp.float32),
            pltpu.VMEM((4096,), jnp.int32),
            pltpu.VMEM((512,), jnp.int32),
            pltpu.VMEM((TIECAP,), jnp.int32),
            pltpu.VMEM((16,), jnp.int32),
            pltpu.SemaphoreType.DMA,
            pltpu.SemaphoreType.DMA,
        ],
        compiler_params=pltpu.CompilerParams(needs_layout_passes=False),
    )
    mask, _, _ = run(x4)
    return mask.reshape(B, N, 1)
